# Initial kernel scaffold; baseline (speedup 1.0000x reference)
import functools

import jax
import jax.numpy as jnp
from jax import lax
from jax.experimental import pallas as pl
from jax.experimental.pallas import tpu as pltpu
from jax.experimental.pallas import tpu_sc as plsc

N = 10000
E = 320000
NC, NS = 2, 16
NW = NC * NS
EPW = E // NW
K = 80
NWIN = EPW // K
RPT = N // NS

_MESH = plsc.VectorSubcoreMesh(core_axis_name="c", subcore_axis_name="s")


def _sc_degrees(src2d, dstn2d, val2, z2):
    @functools.partial(
        pl.kernel,
        mesh=_MESH,
        out_type=jax.ShapeDtypeStruct((NC * 2 * N, 2), jnp.float32),
        scratch_types=[
            pltpu.VMEM((NWIN, K), jnp.int32),
            pltpu.VMEM((NWIN, K), jnp.int32),
            pltpu.VMEM((EPW, 2), jnp.float32),
            pltpu.VMEM_SHARED((2 * N, 2), jnp.float32),
        ],
    )
    def k(src_hbm, dstn_hbm, val_hbm, z_hbm, out_hbm, srcv, dstv, valv, acc):
        cid = lax.axis_index("c")
        sid = lax.axis_index("s")
        wid = sid * NC + cid
        pltpu.sync_copy(src_hbm.at[pl.ds(wid * NWIN, NWIN)], srcv)
        pltpu.sync_copy(dstn_hbm.at[pl.ds(wid * NWIN, NWIN)], dstv)
        pltpu.sync_copy(val_hbm.at[pl.ds(wid * EPW, EPW)], valv)
        @pl.when(sid < 10)
        def _():
            pltpu.sync_copy(z_hbm.at[pl.ds(sid * 2000, 2000)],
                            acc.at[pl.ds(sid * 2000, 2000)])
        plsc.subcore_barrier()

        def win(w, carry):
            pltpu.sync_copy(valv.at[pl.ds(w * K, K)], acc.at[srcv.at[w]],
                            add=True)
            pltpu.sync_copy(valv.at[pl.ds(w * K, K)], acc.at[dstv.at[w]],
                            add=True)
            return carry

        lax.fori_loop(0, NWIN, win, 0)
        plsc.subcore_barrier()

        @pl.when(sid < 10)
        def _():
            pltpu.sync_copy(acc.at[pl.ds(sid * 2000, 2000)],
                            out_hbm.at[pl.ds(cid * 2 * N + sid * 2000, 2000)])

    return k(src2d, dstn2d, val2, z2)


def _sc_layer(gt, src2d, dst2d, ew, zeros, d):
    @functools.partial(
        pl.kernel,
        mesh=_MESH,
        out_type=jax.ShapeDtypeStruct((NC * N, d), jnp.float32),
        scratch_types=[
            pltpu.VMEM((NWIN, K), jnp.int32),
            pltpu.VMEM((NWIN, K), jnp.int32),
            pltpu.VMEM((EPW,), jnp.float32),
            pltpu.VMEM((K, d), jnp.float32),
            pltpu.VMEM_SHARED((N, d), jnp.float32),
            pltpu.SemaphoreType.DMA,
        ],
    )
    def k(gt_hbm, src_hbm, dst_hbm, ew_hbm, z_hbm, out_hbm,
          srcv, dstv, ewv, rows, acc, sem):
        cid = lax.axis_index("c")
        sid = lax.axis_index("s")
        wid = sid * NC + cid
        pltpu.sync_copy(src_hbm.at[pl.ds(wid * NWIN, NWIN)], srcv)
        pltpu.sync_copy(dst_hbm.at[pl.ds(wid * NWIN, NWIN)], dstv)
        pltpu.sync_copy(ew_hbm.at[pl.ds(wid * EPW, EPW)], ewv)
        pltpu.sync_copy(z_hbm.at[pl.ds(sid * RPT, RPT)],
                        acc.at[pl.ds(sid * RPT, RPT)])
        plsc.subcore_barrier()

        def win(w, carry):
            pltpu.async_copy(gt_hbm.at[srcv.at[w]], rows, sem).wait()

            def row(r, c2):
                bidx = jnp.full((16,), w * K + r, jnp.int32)
                wvec = plsc.load_gather(ewv, [bidx])
                for j in range(d // 16):
                    sl = pl.ds(j * 16, 16)
                    rows[r, sl] = rows[r, sl] * wvec
                return c2

            lax.fori_loop(0, K, row, 0)
            pltpu.sync_copy(rows, acc.at[dstv.at[w]], add=True)
            return carry

        lax.fori_loop(0, NWIN, win, 0)
        plsc.subcore_barrier()
        pltpu.sync_copy(acc.at[pl.ds(sid * RPT, RPT)],
                        out_hbm.at[pl.ds(cid * N + sid * RPT, RPT)])

    return k(gt, src2d, dst2d, ew, zeros)


def _tc_first(abp, feats, w0):
    def body(ab_ref, x_ref, w_ref, gt_ref, a_ref, c_ref):
        ab = ab_ref[pl.ds(0, 2 * N), :] + ab_ref[pl.ds(2 * N, 2 * N), :]
        wo = ab[0:N, 0:1]
        do = ab[0:N, 1:2]
        wi = ab[N:2 * N, 0:1]
        di = ab[N:2 * N, 1:2]
        a = lax.rsqrt(jnp.where(wo <= 0.0, 1.0, wo)) * lax.rsqrt(
            jnp.maximum(do, 1.0))
        c = lax.rsqrt(jnp.where(wi <= 0.0, 1.0, wi)) * lax.rsqrt(
            jnp.maximum(di, 1.0))
        a_ref[...] = a
        c_ref[...] = c
        gt_ref[...] = jnp.dot(x_ref[...] * a, w_ref[...],
                              preferred_element_type=jnp.float32)

    return pl.pallas_call(
        body,
        out_shape=(
            jax.ShapeDtypeStruct((N, 128), jnp.float32),
            jax.ShapeDtypeStruct((N, 1), jnp.float32),
            jax.ShapeDtypeStruct((N, 1), jnp.float32),
        ),
    )(abp, feats, w0)


def _tc_mid(pp, c, b, a, w):
    dn = w.shape[1]

    def body(p_ref, c_ref, b_ref, a_ref, w_ref, gt_ref):
        p = p_ref[pl.ds(0, N), :] + p_ref[pl.ds(N, N), :]
        x = jnp.maximum(p * c_ref[...] + b_ref[...], 0.0)
        gt_ref[...] = jnp.dot(x * a_ref[...], w_ref[...],
                              preferred_element_type=jnp.float32)

    return pl.pallas_call(
        body,
        out_shape=jax.ShapeDtypeStruct((N, dn), jnp.float32),
    )(pp, c, b, a, w)


def _tc_final(pp, c, b):
    d = pp.shape[1]

    def body(p_ref, c_ref, b_ref, o_ref):
        p = p_ref[pl.ds(0, N), :] + p_ref[pl.ds(N, N), :]
        o_ref[...] = p * c_ref[...] + b_ref[...]

    return pl.pallas_call(
        body,
        out_shape=jax.ShapeDtypeStruct((N, d), jnp.float32),
    )(pp, c, b)


def kernel(features, edge_index, edge_weight, W0, b0, W1, b1, W2, b2):
    src = edge_index[0]
    dst = edge_index[1]
    src2d = src.reshape(E // K, K)
    dst2d = dst.reshape(E // K, K)
    dstn2d = (dst + N).reshape(E // K, K)
    val2 = jnp.stack([edge_weight, jnp.ones_like(edge_weight)], axis=1)
    z2 = jnp.zeros((2 * N, 2), jnp.float32)
    z128 = jnp.zeros((N, 128), jnp.float32)
    z64 = jnp.zeros((N, 64), jnp.float32)

    abp = _sc_degrees(src2d, dstn2d, val2, z2)
    gt0, a, c = _tc_first(abp, features, W0)
    p0 = _sc_layer(gt0, src2d, dst2d, edge_weight, z128, 128)
    gt1 = _tc_mid(p0, c, b0.reshape(1, 128), a, W1)
    p1 = _sc_layer(gt1, src2d, dst2d, edge_weight, z128, 128)
    gt2 = _tc_mid(p1, c, b1.reshape(1, 128), a, W2)
    p2 = _sc_layer(gt2, src2d, dst2d, edge_weight, z64, 64)
    return _tc_final(p2, c, b2.reshape(1, 64))


# speedup vs baseline: 8.9440x; 8.9440x over previous
import functools

import jax
import jax.numpy as jnp
from jax import lax
from jax.experimental import pallas as pl
from jax.experimental.pallas import tpu as pltpu
from jax.experimental.pallas import tpu_sc as plsc

N = 10000
E = 320000
NC, NS = 2, 16
NW = NC * NS
EPW = E // NW
K = 80
NWIN = EPW // K
RPT = 1000
RPT2 = 2000

_MESH = plsc.VectorSubcoreMesh(core_axis_name="c", subcore_axis_name="s")


def _sc_degrees(src3, dstn3, ew, ones_k, z2n):
    @functools.partial(
        pl.kernel,
        mesh=_MESH,
        compiler_params=pltpu.CompilerParams(
            needs_layout_passes=False, use_tc_tiling_on_sc=False),
        out_type=jax.ShapeDtypeStruct((NC * 2 * 2 * N,), jnp.float32),
        scratch_types=[
            pltpu.VMEM((NWIN, K), jnp.int32),
            pltpu.VMEM((NWIN, K), jnp.int32),
            pltpu.VMEM((EPW,), jnp.float32),
            pltpu.VMEM((K,), jnp.float32),
            pltpu.VMEM((RPT2,), jnp.float32),
            pltpu.VMEM_SHARED((2 * N,), jnp.float32),
            pltpu.VMEM_SHARED((2 * N,), jnp.float32),
        ],
    )
    def k(src_hbm, dstn_hbm, ew_hbm, ones_hbm, z_hbm, out_hbm,
          srcv, dstnv, ewv, onesv, buf, wacc, dacc):
        cid = lax.axis_index("c")
        sid = lax.axis_index("s")
        wid = sid * NC + cid
        pltpu.sync_copy(src_hbm.at[wid], srcv)
        pltpu.sync_copy(dstn_hbm.at[wid], dstnv)
        pltpu.sync_copy(ew_hbm.at[pl.ds(wid * EPW, EPW)], ewv)
        pltpu.sync_copy(ones_hbm, onesv)
        @pl.when(sid < 10)
        def _():
            pltpu.sync_copy(z_hbm.at[pl.ds(sid * RPT2, RPT2)], buf)
            pltpu.sync_copy(buf, wacc.at[pl.ds(sid * RPT2, RPT2)])
            pltpu.sync_copy(buf, dacc.at[pl.ds(sid * RPT2, RPT2)])
        plsc.subcore_barrier()

        def win(w, carry):
            ews = ewv.at[pl.ds(w * K, K)]
            pltpu.sync_copy(ews, wacc.at[srcv.at[w]], add=True)
            pltpu.sync_copy(ews, wacc.at[dstnv.at[w]], add=True)
            pltpu.sync_copy(onesv, dacc.at[srcv.at[w]], add=True)
            pltpu.sync_copy(onesv, dacc.at[dstnv.at[w]], add=True)
            return carry

        lax.fori_loop(0, NWIN, win, 0)
        plsc.subcore_barrier()

        @pl.when(sid < 10)
        def _():
            pltpu.sync_copy(wacc.at[pl.ds(sid * RPT2, RPT2)], buf)
            pltpu.sync_copy(
                buf, out_hbm.at[pl.ds(cid * 4 * N + sid * RPT2, RPT2)])
            pltpu.sync_copy(dacc.at[pl.ds(sid * RPT2, RPT2)], buf)
            pltpu.sync_copy(
                buf, out_hbm.at[pl.ds(cid * 4 * N + 2 * N + sid * RPT2, RPT2)])

    return k(src3, dstn3, ew, ones_k, z2n)


def _sc_edge_scale(st, src, dstn, ew):
    G = EPW // 16

    @functools.partial(
        pl.kernel,
        mesh=_MESH,
        compiler_params=pltpu.CompilerParams(
            needs_layout_passes=False, use_tc_tiling_on_sc=False),
        out_type=jax.ShapeDtypeStruct((E,), jnp.float32),
        scratch_types=[
            pltpu.VMEM((2 * N,), jnp.float32),
            pltpu.VMEM((EPW,), jnp.int32),
            pltpu.VMEM((EPW,), jnp.int32),
            pltpu.VMEM((EPW,), jnp.float32),
            pltpu.VMEM((EPW,), jnp.float32),
        ],
    )
    def k(st_hbm, src_hbm, dstn_hbm, ew_hbm, out_hbm, stv, srcv, dstnv, ewv, mv):
        cid = lax.axis_index("c")
        sid = lax.axis_index("s")
        wid = sid * NC + cid
        pltpu.sync_copy(st_hbm, stv)
        pltpu.sync_copy(src_hbm.at[pl.ds(wid * EPW, EPW)], srcv)
        pltpu.sync_copy(dstn_hbm.at[pl.ds(wid * EPW, EPW)], dstnv)
        pltpu.sync_copy(ew_hbm.at[pl.ds(wid * EPW, EPW)], ewv)

        def step(g, carry):
            sl = pl.ds(g * 16, 16)
            s16 = plsc.load_gather(stv, [srcv[sl]])
            t16 = plsc.load_gather(stv, [dstnv[sl]])
            mv[sl] = ewv[sl] * s16 * t16
            return carry

        lax.fori_loop(0, G, step, 0)
        pltpu.sync_copy(mv, out_hbm.at[pl.ds(wid * EPW, EPW)])

    return k(st, src, dstn, ew)


def _sc_layer(gt, src, dst3, m, zeros, d):
    @functools.partial(
        pl.kernel,
        mesh=_MESH,
        compiler_params=pltpu.CompilerParams(
            needs_layout_passes=False, use_tc_tiling_on_sc=False),
        out_type=jax.ShapeDtypeStruct((NC * N, d), jnp.float32),
        scratch_types=[
            pltpu.VMEM((EPW,), jnp.int32),
            pltpu.VMEM((NWIN, K), jnp.int32),
            pltpu.VMEM((EPW,), jnp.float32),
            pltpu.VMEM((K, d), jnp.float32),
            pltpu.SemaphoreType.DMA,
            pltpu.VMEM_SHARED((N, d), jnp.float32),
        ],
    )
    def k(gt_hbm, src_hbm, dst_hbm, m_hbm, z_hbm, out_hbm,
          srcv, dstv, mv, rows, sem, acc):
        cid = lax.axis_index("c")
        sid = lax.axis_index("s")
        wid = sid * NC + cid
        pltpu.sync_copy(src_hbm.at[pl.ds(wid * EPW, EPW)], srcv)
        pltpu.sync_copy(dst_hbm.at[wid], dstv)
        pltpu.sync_copy(m_hbm.at[pl.ds(wid * EPW, EPW)], mv)
        @pl.when(sid < 10)
        def _():
            pltpu.sync_copy(z_hbm.at[pl.ds(sid * RPT, RPT)],
                            acc.at[pl.ds(sid * RPT, RPT)])
        plsc.subcore_barrier()

        def win(w, carry):
            pltpu.async_copy(
                gt_hbm.at[srcv.at[pl.ds(w * K, K)]], rows, sem).wait()

            def row(r, c2):
                bidx = jnp.full((16,), w * K + r, jnp.int32)
                wvec = plsc.load_gather(mv, [bidx])
                for j in range(d // 16):
                    sl = pl.ds(j * 16, 16)
                    rows[r, sl] = rows[r, sl] * wvec
                return c2

            lax.fori_loop(0, K, row, 0)
            pltpu.sync_copy(rows, acc.at[dstv.at[w]], add=True)
            return carry

        lax.fori_loop(0, NWIN, win, 0)
        plsc.subcore_barrier()

        @pl.when(sid < 10)
        def _():
            pltpu.sync_copy(acc.at[pl.ds(sid * RPT, RPT)],
                            out_hbm.at[pl.ds(cid * N + sid * RPT, RPT)])

    return k(gt, src, dst3, m, zeros)


def _tc_first(d4, feats, w0):
    def body(d_ref, x_ref, w_ref, st_ref, gt_ref):
        wdeg = d_ref[0, :] + d_ref[2, :]
        deg = d_ref[1, :] + d_ref[3, :]
        wdeg = jnp.where(wdeg <= 0.0, 1.0, wdeg)
        deg = jnp.maximum(deg, 1.0)
        st_ref[...] = lax.rsqrt(wdeg) * lax.rsqrt(deg)
        gt_ref[...] = jnp.dot(x_ref[...], w_ref[...],
                              preferred_element_type=jnp.float32)

    return pl.pallas_call(
        body,
        out_shape=(
            jax.ShapeDtypeStruct((2 * N,), jnp.float32),
            jax.ShapeDtypeStruct((N, 128), jnp.float32),
        ),
    )(d4, feats, w0)


def _tc_mid(pp, b, w):
    dn = w.shape[1]

    def body(p_ref, b_ref, w_ref, gt_ref):
        p = p_ref[pl.ds(0, N), :] + p_ref[pl.ds(N, N), :]
        x = jnp.maximum(p + b_ref[...], 0.0)
        gt_ref[...] = jnp.dot(x, w_ref[...],
                              preferred_element_type=jnp.float32)

    return pl.pallas_call(
        body,
        out_shape=jax.ShapeDtypeStruct((N, dn), jnp.float32),
    )(pp, b, w)


def _tc_final(pp, b):
    d = b.shape[1]

    def body(p_ref, b_ref, o_ref):
        o_ref[...] = (p_ref[pl.ds(0, N), pl.ds(0, d)]
                      + p_ref[pl.ds(N, N), pl.ds(0, d)] + b_ref[...])

    return pl.pallas_call(
        body,
        out_shape=jax.ShapeDtypeStruct((N, d), jnp.float32),
    )(pp, b)


def kernel(features, edge_index, edge_weight, W0, b0, W1, b1, W2, b2):
    src = edge_index[0]
    dst = edge_index[1]
    dstn = dst + N
    src3 = src.reshape(NW, NWIN, K)
    dst3 = dst.reshape(NW, NWIN, K)
    dstn3 = dstn.reshape(NW, NWIN, K)
    ones_k = jnp.ones((K,), jnp.float32)
    z2n = jnp.zeros((2 * N,), jnp.float32)
    z128 = jnp.zeros((N, 128), jnp.float32)
    W2p = jnp.pad(W2, ((0, 0), (0, 128 - W2.shape[1])))

    dtab = _sc_degrees(src3, dstn3, edge_weight, ones_k, z2n)
    st, gt0 = _tc_first(dtab.reshape(4, 2 * N), features, W0)
    m = _sc_edge_scale(st, src, dstn, edge_weight)
    p0 = _sc_layer(gt0, src, dst3, m, z128, 128)
    gt1 = _tc_mid(p0, b0.reshape(1, 128), W1)
    p1 = _sc_layer(gt1, src, dst3, m, z128, 128)
    gt2 = _tc_mid(p1, b1.reshape(1, 128), W2p)
    p2 = _sc_layer(gt2, src, dst3, m, z128, 128)
    return _tc_final(p2, b2.reshape(1, 64))


# speedup vs baseline: 14.2364x; 1.5917x over previous
import functools

import jax
import jax.numpy as jnp
from jax import lax
from jax.experimental import pallas as pl
from jax.experimental.pallas import tpu as pltpu
from jax.experimental.pallas import tpu_sc as plsc

N = 10000
E = 320000
NC, NS = 2, 16
NW = NC * NS
EPW = E // NW
K = 80
NWIN = EPW // K
RPT = 1000
RPT2 = 2000

_MESH = plsc.VectorSubcoreMesh(core_axis_name="c", subcore_axis_name="s")


def _sc_degrees(src3, dstn3, ew, ones_k, z2n):
    @functools.partial(
        pl.kernel,
        mesh=_MESH,
        compiler_params=pltpu.CompilerParams(
            needs_layout_passes=False, use_tc_tiling_on_sc=False),
        out_type=jax.ShapeDtypeStruct((NC * 2 * 2 * N,), jnp.float32),
        scratch_types=[
            pltpu.VMEM((NWIN, K), jnp.int32),
            pltpu.VMEM((NWIN, K), jnp.int32),
            pltpu.VMEM((EPW,), jnp.float32),
            pltpu.VMEM((K,), jnp.float32),
            pltpu.VMEM((RPT2,), jnp.float32),
            pltpu.VMEM_SHARED((2 * N,), jnp.float32),
            pltpu.VMEM_SHARED((2 * N,), jnp.float32),
        ],
    )
    def k(src_hbm, dstn_hbm, ew_hbm, ones_hbm, z_hbm, out_hbm,
          srcv, dstnv, ewv, onesv, buf, wacc, dacc):
        cid = lax.axis_index("c")
        sid = lax.axis_index("s")
        wid = sid * NC + cid
        pltpu.sync_copy(src_hbm.at[wid], srcv)
        pltpu.sync_copy(dstn_hbm.at[wid], dstnv)
        pltpu.sync_copy(ew_hbm.at[pl.ds(wid * EPW, EPW)], ewv)
        pltpu.sync_copy(ones_hbm, onesv)
        @pl.when(sid < 10)
        def _():
            pltpu.sync_copy(z_hbm.at[pl.ds(sid * RPT2, RPT2)], buf)
            pltpu.sync_copy(buf, wacc.at[pl.ds(sid * RPT2, RPT2)])
            pltpu.sync_copy(buf, dacc.at[pl.ds(sid * RPT2, RPT2)])
        plsc.subcore_barrier()

        def win(w, carry):
            ews = ewv.at[pl.ds(w * K, K)]
            pltpu.sync_copy(ews, wacc.at[srcv.at[w]], add=True)
            pltpu.sync_copy(ews, wacc.at[dstnv.at[w]], add=True)
            pltpu.sync_copy(onesv, dacc.at[srcv.at[w]], add=True)
            pltpu.sync_copy(onesv, dacc.at[dstnv.at[w]], add=True)
            return carry

        lax.fori_loop(0, NWIN, win, 0)
        plsc.subcore_barrier()

        @pl.when(sid < 10)
        def _():
            pltpu.sync_copy(wacc.at[pl.ds(sid * RPT2, RPT2)], buf)
            pltpu.sync_copy(
                buf, out_hbm.at[pl.ds(cid * 4 * N + sid * RPT2, RPT2)])
            pltpu.sync_copy(dacc.at[pl.ds(sid * RPT2, RPT2)], buf)
            pltpu.sync_copy(
                buf, out_hbm.at[pl.ds(cid * 4 * N + 2 * N + sid * RPT2, RPT2)])

    return k(src3, dstn3, ew, ones_k, z2n)


def _sc_layer(gt, src, dst3, m, zeros, d):
    @functools.partial(
        pl.kernel,
        mesh=_MESH,
        compiler_params=pltpu.CompilerParams(
            needs_layout_passes=False, use_tc_tiling_on_sc=False),
        out_type=jax.ShapeDtypeStruct((NC * N, d), jnp.float32),
        scratch_types=[
            pltpu.VMEM((EPW,), jnp.int32),
            pltpu.VMEM((NWIN, K), jnp.int32),
            pltpu.VMEM((EPW,), jnp.float32),
            pltpu.VMEM((K, d), jnp.float32),
            pltpu.VMEM((K, d), jnp.float32),
            pltpu.SemaphoreType.DMA,
            pltpu.SemaphoreType.DMA,
            pltpu.VMEM_SHARED((N, d), jnp.float32),
        ],
    )
    def k(gt_hbm, src_hbm, dst_hbm, m_hbm, z_hbm, out_hbm,
          srcv, dstv, mv, rows0, rows1, sem0, sem1, acc):
        cid = lax.axis_index("c")
        sid = lax.axis_index("s")
        wid = sid * NC + cid
        pltpu.sync_copy(src_hbm.at[pl.ds(wid * EPW, EPW)], srcv)
        pltpu.sync_copy(dst_hbm.at[wid], dstv)
        pltpu.sync_copy(m_hbm.at[pl.ds(wid * EPW, EPW)], mv)
        @pl.when(sid < 10)
        def _():
            pltpu.sync_copy(z_hbm.at[pl.ds(sid * RPT, RPT)],
                            acc.at[pl.ds(sid * RPT, RPT)])
        plsc.subcore_barrier()

        def start(w, rows, sem):
            pltpu.async_copy(gt_hbm.at[srcv.at[pl.ds(w * K, K)]], rows, sem)

        def finish(w, rows, sem):
            pltpu.make_async_copy(
                gt_hbm.at[srcv.at[pl.ds(w * K, K)]], rows, sem).wait()

            def row(r, c2):
                bidx = jnp.full((16,), w * K + r, jnp.int32)
                wvec = plsc.load_gather(mv, [bidx])
                for j in range(d // 16):
                    sl = pl.ds(j * 16, 16)
                    rows[r, sl] = rows[r, sl] * wvec
                return c2

            lax.fori_loop(0, K, row, 0)
            pltpu.sync_copy(rows, acc.at[dstv.at[w]], add=True)

        start(0, rows0, sem0)

        def pair(g, carry):
            w = 2 * g
            start(w + 1, rows1, sem1)
            finish(w, rows0, sem0)
            start(w + 2, rows0, sem0)
            finish(w + 1, rows1, sem1)
            return carry

        lax.fori_loop(0, (NWIN - 1) // 2, pair, 0)
        finish(NWIN - 1, rows0, sem0)
        plsc.subcore_barrier()

        @pl.when(sid < 10)
        def _():
            pltpu.sync_copy(acc.at[pl.ds(sid * RPT, RPT)],
                            out_hbm.at[pl.ds(cid * N + sid * RPT, RPT)])

    return k(gt, src, dst3, m, zeros)


def _tc_first(d4, feats, w0):
    def body(d_ref, x_ref, w_ref, s_ref, t_ref, gt_ref):
        wdeg = d_ref[0, :] + d_ref[2, :]
        deg = d_ref[1, :] + d_ref[3, :]
        wdeg = jnp.where(wdeg <= 0.0, 1.0, wdeg)
        deg = jnp.maximum(deg, 1.0)
        st = lax.rsqrt(wdeg) * lax.rsqrt(deg)
        s = st[:N].reshape(N, 1)
        t = st[N:].reshape(N, 1)
        s_ref[...] = s
        t_ref[...] = t
        gt_ref[...] = jnp.dot(x_ref[...] * s, w_ref[...],
                              preferred_element_type=jnp.float32)

    return pl.pallas_call(
        body,
        out_shape=(
            jax.ShapeDtypeStruct((N, 1), jnp.float32),
            jax.ShapeDtypeStruct((N, 1), jnp.float32),
            jax.ShapeDtypeStruct((N, 128), jnp.float32),
        ),
    )(d4, feats, w0)


def _tc_mid(pp, b, w, scol, tcol):
    dn = w.shape[1]

    def body(p_ref, b_ref, w_ref, s_ref, t_ref, gt_ref):
        p = p_ref[pl.ds(0, N), :] + p_ref[pl.ds(N, N), :]
        x = jnp.maximum(p * t_ref[...] + b_ref[...], 0.0)
        gt_ref[...] = jnp.dot(x * s_ref[...], w_ref[...],
                              preferred_element_type=jnp.float32)

    return pl.pallas_call(
        body,
        out_shape=jax.ShapeDtypeStruct((N, dn), jnp.float32),
    )(pp, b, w, scol, tcol)


def _tc_final(pp, b, tcol):
    d = b.shape[1]

    def body(p_ref, b_ref, t_ref, o_ref):
        p = p_ref[pl.ds(0, N), pl.ds(0, d)] + p_ref[pl.ds(N, N), pl.ds(0, d)]
        o_ref[...] = p * t_ref[...] + b_ref[...]

    return pl.pallas_call(
        body,
        out_shape=jax.ShapeDtypeStruct((N, d), jnp.float32),
    )(pp, b, tcol)


def kernel(features, edge_index, edge_weight, W0, b0, W1, b1, W2, b2):
    src = edge_index[0]
    dst = edge_index[1]
    src3 = src.reshape(NW, NWIN, K)
    dst3 = dst.reshape(NW, NWIN, K)
    dstn3 = (dst + N).reshape(NW, NWIN, K)
    ones_k = jnp.ones((K,), jnp.float32)
    z2n = jnp.zeros((2 * N,), jnp.float32)
    z128 = jnp.zeros((N, 128), jnp.float32)
    W2p = jnp.pad(W2, ((0, 0), (0, 128 - W2.shape[1])))

    dtab = _sc_degrees(src3, dstn3, edge_weight, ones_k, z2n)
    scol, tcol, gt0 = _tc_first(dtab.reshape(4, 2 * N), features, W0)
    p0 = _sc_layer(gt0, src, dst3, edge_weight, z128, 128)
    gt1 = _tc_mid(p0, b0.reshape(1, 128), W1, scol, tcol)
    p1 = _sc_layer(gt1, src, dst3, edge_weight, z128, 128)
    gt2 = _tc_mid(p1, b1.reshape(1, 128), W2p, scol, tcol)
    p2 = _sc_layer(gt2, src, dst3, edge_weight, z128, 128)
    return _tc_final(p2, b2.reshape(1, 64), tcol)


# speedup vs baseline: 16.6807x; 1.1717x over previous
import functools

import jax
import jax.numpy as jnp
from jax import lax
from jax.experimental import pallas as pl
from jax.experimental.pallas import tpu as pltpu
from jax.experimental.pallas import tpu_sc as plsc

N = 10000
E = 320000
NC, NS = 2, 16
NW = NC * NS
EPW = E // NW
K = 80
NWIN = EPW // K
RPT = 1000
RPT2 = 2000

_MESH = plsc.VectorSubcoreMesh(core_axis_name="c", subcore_axis_name="s")


def _sc_degrees(src3, dstn3, ew, ones_k, z2n):
    @functools.partial(
        pl.kernel,
        mesh=_MESH,
        compiler_params=pltpu.CompilerParams(
            needs_layout_passes=False, use_tc_tiling_on_sc=False),
        out_type=jax.ShapeDtypeStruct((NC * 2 * 2 * N,), jnp.float32),
        scratch_types=[
            pltpu.VMEM((NWIN, K), jnp.int32),
            pltpu.VMEM((NWIN, K), jnp.int32),
            pltpu.VMEM((EPW,), jnp.float32),
            pltpu.VMEM((K,), jnp.float32),
            pltpu.VMEM((RPT2,), jnp.float32),
            pltpu.VMEM_SHARED((2 * N,), jnp.float32),
            pltpu.VMEM_SHARED((2 * N,), jnp.float32),
        ],
    )
    def k(src_hbm, dstn_hbm, ew_hbm, ones_hbm, z_hbm, out_hbm,
          srcv, dstnv, ewv, onesv, buf, wacc, dacc):
        cid = lax.axis_index("c")
        sid = lax.axis_index("s")
        wid = sid * NC + cid
        pltpu.sync_copy(src_hbm.at[wid], srcv)
        pltpu.sync_copy(dstn_hbm.at[wid], dstnv)
        pltpu.sync_copy(ew_hbm.at[pl.ds(wid * EPW, EPW)], ewv)
        pltpu.sync_copy(ones_hbm, onesv)
        @pl.when(sid < 10)
        def _():
            pltpu.sync_copy(z_hbm.at[pl.ds(sid * RPT2, RPT2)], buf)
            pltpu.sync_copy(buf, wacc.at[pl.ds(sid * RPT2, RPT2)])
            pltpu.sync_copy(buf, dacc.at[pl.ds(sid * RPT2, RPT2)])
        plsc.subcore_barrier()

        def win(w, carry):
            ews = ewv.at[pl.ds(w * K, K)]
            pltpu.sync_copy(ews, wacc.at[srcv.at[w]], add=True)
            pltpu.sync_copy(ews, wacc.at[dstnv.at[w]], add=True)
            pltpu.sync_copy(onesv, dacc.at[srcv.at[w]], add=True)
            pltpu.sync_copy(onesv, dacc.at[dstnv.at[w]], add=True)
            return carry

        lax.fori_loop(0, NWIN, win, 0)
        plsc.subcore_barrier()

        @pl.when(sid < 10)
        def _():
            pltpu.sync_copy(wacc.at[pl.ds(sid * RPT2, RPT2)], buf)
            pltpu.sync_copy(
                buf, out_hbm.at[pl.ds(cid * 4 * N + sid * RPT2, RPT2)])
            pltpu.sync_copy(dacc.at[pl.ds(sid * RPT2, RPT2)], buf)
            pltpu.sync_copy(
                buf, out_hbm.at[pl.ds(cid * 4 * N + 2 * N + sid * RPT2, RPT2)])

    return k(src3, dstn3, ew, ones_k, z2n)


def _sc_layer(gt, src, dst3, m, zeros, d):
    @functools.partial(
        pl.kernel,
        mesh=_MESH,
        compiler_params=pltpu.CompilerParams(
            needs_layout_passes=False, use_tc_tiling_on_sc=False),
        out_type=jax.ShapeDtypeStruct((NC * N, d), jnp.float32),
        scratch_types=[
            pltpu.VMEM((EPW,), jnp.int32),
            pltpu.VMEM((NWIN, K), jnp.int32),
            pltpu.VMEM((EPW,), jnp.float32),
            pltpu.VMEM((K, d), jnp.float32),
            pltpu.VMEM((K, d), jnp.float32),
            pltpu.SemaphoreType.DMA,
            pltpu.SemaphoreType.DMA,
            pltpu.VMEM_SHARED((N, d), jnp.float32),
        ],
    )
    def k(gt_hbm, src_hbm, dst_hbm, m_hbm, z_hbm, out_hbm,
          srcv, dstv, mv, rows0, rows1, sem0, sem1, acc):
        cid = lax.axis_index("c")
        sid = lax.axis_index("s")
        wid = sid * NC + cid
        pltpu.sync_copy(src_hbm.at[pl.ds(wid * EPW, EPW)], srcv)
        pltpu.sync_copy(dst_hbm.at[wid], dstv)
        pltpu.sync_copy(m_hbm.at[pl.ds(wid * EPW, EPW)], mv)
        @pl.when(sid < 10)
        def _():
            pltpu.sync_copy(z_hbm.at[pl.ds(sid * RPT, RPT)],
                            acc.at[pl.ds(sid * RPT, RPT)])
        plsc.subcore_barrier()

        def start(w, rows, sem):
            pltpu.async_copy(gt_hbm.at[srcv.at[pl.ds(w * K, K)]], rows, sem)

        def finish(w, rows, sem):
            pltpu.make_async_copy(
                gt_hbm.at[srcv.at[pl.ds(w * K, K)]], rows, sem).wait()

            @plsc.parallel_loop(0, K, unroll=4)
            def row(r):
                bidx = jnp.full((16,), w * K + r, jnp.int32)
                wvec = plsc.load_gather(mv, [bidx])
                for j in range(d // 16):
                    sl = pl.ds(j * 16, 16)
                    rows[r, sl] = rows[r, sl] * wvec
            pltpu.sync_copy(rows, acc.at[dstv.at[w]], add=True)

        start(0, rows0, sem0)

        def pair(g, carry):
            w = 2 * g
            start(w + 1, rows1, sem1)
            finish(w, rows0, sem0)
            start(w + 2, rows0, sem0)
            finish(w + 1, rows1, sem1)
            return carry

        lax.fori_loop(0, (NWIN - 1) // 2, pair, 0)
        finish(NWIN - 1, rows0, sem0)
        plsc.subcore_barrier()

        @pl.when(sid < 10)
        def _():
            pltpu.sync_copy(acc.at[pl.ds(sid * RPT, RPT)],
                            out_hbm.at[pl.ds(cid * N + sid * RPT, RPT)])

    return k(gt, src, dst3, m, zeros)


def _tc_first(d4, feats, w0):
    def body(d_ref, x_ref, w_ref, s_ref, t_ref, gt_ref):
        wdeg = d_ref[0, :] + d_ref[2, :]
        deg = d_ref[1, :] + d_ref[3, :]
        wdeg = jnp.where(wdeg <= 0.0, 1.0, wdeg)
        deg = jnp.maximum(deg, 1.0)
        st = lax.rsqrt(wdeg) * lax.rsqrt(deg)
        s = st[:N].reshape(N, 1)
        t = st[N:].reshape(N, 1)
        s_ref[...] = s
        t_ref[...] = t
        gt_ref[...] = jnp.dot(x_ref[...] * s, w_ref[...],
                              preferred_element_type=jnp.float32)

    return pl.pallas_call(
        body,
        out_shape=(
            jax.ShapeDtypeStruct((N, 1), jnp.float32),
            jax.ShapeDtypeStruct((N, 1), jnp.float32),
            jax.ShapeDtypeStruct((N, 128), jnp.float32),
        ),
    )(d4, feats, w0)


def _tc_mid(pp, b, w, scol, tcol):
    dn = w.shape[1]

    def body(p_ref, b_ref, w_ref, s_ref, t_ref, gt_ref):
        p = p_ref[pl.ds(0, N), :] + p_ref[pl.ds(N, N), :]
        x = jnp.maximum(p * t_ref[...] + b_ref[...], 0.0)
        gt_ref[...] = jnp.dot(x * s_ref[...], w_ref[...],
                              preferred_element_type=jnp.float32)

    return pl.pallas_call(
        body,
        out_shape=jax.ShapeDtypeStruct((N, dn), jnp.float32),
    )(pp, b, w, scol, tcol)


def _tc_final(pp, b, tcol):
    d = b.shape[1]

    def body(p_ref, b_ref, t_ref, o_ref):
        p = p_ref[pl.ds(0, N), pl.ds(0, d)] + p_ref[pl.ds(N, N), pl.ds(0, d)]
        o_ref[...] = p * t_ref[...] + b_ref[...]

    return pl.pallas_call(
        body,
        out_shape=jax.ShapeDtypeStruct((N, d), jnp.float32),
    )(pp, b, tcol)


def kernel(features, edge_index, edge_weight, W0, b0, W1, b1, W2, b2):
    src = edge_index[0]
    dst = edge_index[1]
    src3 = src.reshape(NW, NWIN, K)
    dst3 = dst.reshape(NW, NWIN, K)
    dstn3 = (dst + N).reshape(NW, NWIN, K)
    ones_k = jnp.ones((K,), jnp.float32)
    z2n = jnp.zeros((2 * N,), jnp.float32)
    z128 = jnp.zeros((N, 128), jnp.float32)
    W2p = jnp.pad(W2, ((0, 0), (0, 128 - W2.shape[1])))

    dtab = _sc_degrees(src3, dstn3, edge_weight, ones_k, z2n)
    scol, tcol, gt0 = _tc_first(dtab.reshape(4, 2 * N), features, W0)
    p0 = _sc_layer(gt0, src, dst3, edge_weight, z128, 128)
    gt1 = _tc_mid(p0, b0.reshape(1, 128), W1, scol, tcol)
    p1 = _sc_layer(gt1, src, dst3, edge_weight, z128, 128)
    gt2 = _tc_mid(p1, b1.reshape(1, 128), W2p, scol, tcol)
    p2 = _sc_layer(gt2, src, dst3, edge_weight, z128, 128)
    return _tc_final(p2, b2.reshape(1, 64), tcol)
